# Initial kernel scaffold; baseline (speedup 1.0000x reference)
import jax
import jax.numpy as jnp
from jax import lax
from jax.experimental import pallas as pl
from jax.experimental.pallas import tpu as pltpu
from jax.experimental.pallas import tpu_sc as plsc

_N = 10000
_E = 320000
_D = 128
_R = 4

_NC = 2
_NS = 16
_NW = _NC * _NS
_EPW = _E // _NW
_C = 80
_NCHUNK = _EPW // _C
_ROWS_PER_SUB = _N // _NS
_OB = 125

_F32 = jnp.float32
_HI = lax.Precision.HIGHEST


def _proj_body(x_ref, wrel_ref, w1w_ref, mlpw_ref, w1b_ref, mlpb_ref,
               y_ref, wc_ref, bc_ref):
    i = pl.program_id(0)

    @pl.when(i == 0)
    def _():
        for k in range(_R):
            wc_ref[k] = jax.lax.dot(wrel_ref[k], mlpw_ref[...],
                                    precision=_HI, preferred_element_type=_F32)
        wc_ref[_R] = jax.lax.dot(w1w_ref[...], mlpw_ref[...],
                                 precision=_HI, preferred_element_type=_F32)
        bc_ref[...] = jax.lax.dot(w1b_ref[...], mlpw_ref[...],
                                  precision=_HI, preferred_element_type=_F32) \
            + mlpb_ref[...]

    xb = x_ref[...]
    for k in range(_R + 1):
        yk = jax.lax.dot(xb, wc_ref[k], precision=_HI,
                         preferred_element_type=_F32)
        if k == _R:
            yk = yk + bc_ref[...]
        y_ref[k] = yk


def _project(x, W_rel, w1_W, mlp_W, w1_b, mlp_b, bn):
    nb = _N // bn
    return pl.pallas_call(
        _proj_body,
        grid=(nb,),
        in_specs=[
            pl.BlockSpec((bn, _D), lambda i: (i, 0)),
            pl.BlockSpec((_R, _D, _D), lambda i: (0, 0, 0)),
            pl.BlockSpec((_D, _D), lambda i: (0, 0)),
            pl.BlockSpec((_D, _D), lambda i: (0, 0)),
            pl.BlockSpec((1, _D), lambda i: (0, 0)),
            pl.BlockSpec((1, _D), lambda i: (0, 0)),
        ],
        out_specs=pl.BlockSpec((_R + 1, bn, _D), lambda i: (0, i, 0)),
        out_shape=jax.ShapeDtypeStruct((_R + 1, _N, _D), _F32),
        scratch_shapes=[
            pltpu.VMEM((_R + 1, _D, _D), _F32),
            pltpu.VMEM((1, _D), _F32),
        ],
    )(x, W_rel, w1_W, mlp_W, w1_b.reshape(1, _D), mlp_b.reshape(1, _D))


def _edge_body(y_hbm, g_hbm, d_hbm, out_hbm,
               gidx, didx, rows0, rows1, obuf, acc, sem0, sem1):
    c = lax.axis_index("c")
    s = lax.axis_index("s")
    wid = s * _NC + c
    row_base = wid * _NCHUNK

    pltpu.sync_copy(g_hbm.at[pl.ds(row_base, _NCHUNK)], gidx)
    pltpu.sync_copy(d_hbm.at[pl.ds(row_base, _NCHUNK)], didx)

    zv = jnp.zeros((16,), _F32)

    @pl.loop(0, _OB)
    def _(i):
        for k in range(_D // 16):
            obuf[i, pl.ds(k * 16, 16)] = zv

    for k in range(_ROWS_PER_SUB // _OB):
        pltpu.sync_copy(obuf, acc.at[pl.ds(s * _ROWS_PER_SUB + k * _OB, _OB)])
    plsc.subcore_barrier()

    pltpu.async_copy(y_hbm.at[gidx.at[0]], rows0, sem0)

    @pl.loop(0, _NCHUNK - 1, step=2)
    def _(i):
        pltpu.make_async_copy(y_hbm.at[gidx.at[0]], rows0, sem0).wait()
        pltpu.async_copy(y_hbm.at[gidx.at[i + 1]], rows1, sem1)
        pltpu.sync_copy(rows0, acc.at[didx.at[i]], add=True)
        pltpu.make_async_copy(y_hbm.at[gidx.at[0]], rows1, sem1).wait()
        pltpu.async_copy(y_hbm.at[gidx.at[i + 2]], rows0, sem0)
        pltpu.sync_copy(rows1, acc.at[didx.at[i + 1]], add=True)

    pltpu.make_async_copy(y_hbm.at[gidx.at[0]], rows0, sem0).wait()
    pltpu.sync_copy(rows0, acc.at[didx.at[_NCHUNK - 1]], add=True)
    plsc.subcore_barrier()

    for k in range(_ROWS_PER_SUB // _OB):
        off = s * _ROWS_PER_SUB + k * _OB
        pltpu.sync_copy(acc.at[pl.ds(off, _OB)], obuf)
        pltpu.sync_copy(obuf, out_hbm.at[pl.ds(c * _N + off, _OB)])


def _edge_aggregate(y_flat, g2d, d2d):
    mesh = plsc.VectorSubcoreMesh(core_axis_name="c", subcore_axis_name="s")
    kern = pl.kernel(
        _edge_body,
        out_type=jax.ShapeDtypeStruct((_NC * _N, _D), _F32),
        mesh=mesh,
        scratch_types=[
            pltpu.VMEM((_NCHUNK, _C), jnp.int32),
            pltpu.VMEM((_NCHUNK, _C), jnp.int32),
            pltpu.VMEM((_C, _D), _F32),
            pltpu.VMEM((_C, _D), _F32),
            pltpu.VMEM((_OB, _D), _F32),
            pltpu.VMEM_SHARED((_N, _D), _F32),
            pltpu.SemaphoreType.DMA,
            pltpu.SemaphoreType.DMA,
        ],
    )
    return kern(y_flat, g2d, d2d)


def _combine_body(y_ref, p_ref, out_ref):
    out_ref[...] = y_ref[0] + p_ref[0] + p_ref[1]


def _combine(y5, partials, bn):
    nb = _N // bn
    return pl.pallas_call(
        _combine_body,
        grid=(nb,),
        in_specs=[
            pl.BlockSpec((1, bn, _D), lambda i: (_R, i, 0)),
            pl.BlockSpec((_NC, bn, _D), lambda i: (0, i, 0)),
        ],
        out_specs=pl.BlockSpec((bn, _D), lambda i: (i, 0)),
        out_shape=jax.ShapeDtypeStruct((_N, _D), _F32),
    )(y5, partials)


@jax.jit
def kernel(x, edge_index, etype, W_rel, w1_W, w1_b, mlp_W, mlp_b):
    src = edge_index[0]
    dst = edge_index[1]
    g = (etype * _N + src).reshape(_E // _C, _C)
    d2d = dst.reshape(_E // _C, _C)

    y5 = _project(x, W_rel, w1_W, mlp_W, w1_b, mlp_b, bn=1000)
    y_flat = y5.reshape((_R + 1) * _N, _D)
    partials = _edge_aggregate(y_flat, g, d2d)
    return _combine(y5, partials.reshape(_NC, _N, _D), bn=2000)


# speedup vs baseline: 8.8647x; 8.8647x over previous
import jax
import jax.numpy as jnp
from jax import lax
from jax.experimental import pallas as pl
from jax.experimental.pallas import tpu as pltpu
from jax.experimental.pallas import tpu_sc as plsc

_N = 10000
_E = 320000
_D = 128
_R = 4

_NC = 2
_NS = 16
_NW = _NC * _NS
_EPW = _E // _NW
_C = 40
_NCHUNK = _EPW // _C
_NSTAGE = 2
_SCHUNK = _NCHUNK // _NSTAGE
_WSTRIDE = 624
_WIN = 640

_F32 = jnp.float32
_HI = lax.Precision.HIGHEST


def _proj_body(x_ref, wrel_ref, w1w_ref, mlpw_ref, w1b_ref, mlpb_ref,
               y_ref, wc_ref, bc_ref):
    i = pl.program_id(0)

    @pl.when(i == 0)
    def _():
        for k in range(_R):
            wc_ref[k] = jax.lax.dot(wrel_ref[k], mlpw_ref[...],
                                    precision=_HI, preferred_element_type=_F32)
        wc_ref[_R] = jax.lax.dot(w1w_ref[...], mlpw_ref[...],
                                 precision=_HI, preferred_element_type=_F32)
        bc_ref[...] = jax.lax.dot(w1b_ref[...], mlpw_ref[...],
                                  precision=_HI, preferred_element_type=_F32) \
            + mlpb_ref[...]

    xb = x_ref[...]
    for k in range(_R + 1):
        yk = jax.lax.dot(xb, wc_ref[k], precision=_HI,
                         preferred_element_type=_F32)
        if k == _R:
            yk = yk + bc_ref[...]
        y_ref[k] = yk


def _project(x, W_rel, w1_W, mlp_W, w1_b, mlp_b, bn):
    nb = _N // bn
    return pl.pallas_call(
        _proj_body,
        grid=(nb,),
        in_specs=[
            pl.BlockSpec((bn, _D), lambda i: (i, 0)),
            pl.BlockSpec((_R, _D, _D), lambda i: (0, 0, 0)),
            pl.BlockSpec((_D, _D), lambda i: (0, 0)),
            pl.BlockSpec((_D, _D), lambda i: (0, 0)),
            pl.BlockSpec((1, _D), lambda i: (0, 0)),
            pl.BlockSpec((1, _D), lambda i: (0, 0)),
        ],
        out_specs=pl.BlockSpec((_R + 1, bn, _D), lambda i: (0, i, 0)),
        out_shape=jax.ShapeDtypeStruct((_R + 1, _N, _D), _F32),
        scratch_shapes=[
            pltpu.VMEM((_R + 1, _D, _D), _F32),
            pltpu.VMEM((1, _D), _F32),
        ],
    )(x, W_rel, w1_W, mlp_W, w1_b.reshape(1, _D), mlp_b.reshape(1, _D))


def _edge_body(y_hbm, g_hbm, d_hbm, out_hbm,
               gidx, didx, rows0, rows1, acc, sem0, sem1):
    c = lax.axis_index("c")
    s = lax.axis_index("s")
    wid = s * _NC + c

    zv = jnp.zeros((16,), _F32)

    @pl.loop(0, _C)
    def _(i):
        for k in range(_D // 16):
            rows0[i, pl.ds(k * 16, 16)] = zv

    for j in range(_WIN // _C):
        pltpu.sync_copy(rows0, acc.at[pl.ds(s * _WSTRIDE + j * _C, _C)])
    plsc.subcore_barrier()

    for h in range(_NSTAGE):
        pltpu.sync_copy(g_hbm.at[wid * _NSTAGE + h], gidx)
        pltpu.sync_copy(d_hbm.at[wid * _NSTAGE + h], didx)
        pltpu.async_copy(y_hbm.at[gidx.at[0]], rows0, sem0)

        @pl.loop(0, _SCHUNK, step=2)
        def _(i):
            pltpu.make_async_copy(y_hbm.at[gidx.at[0]], rows0, sem0).wait()

            @pl.when(i + 1 < _SCHUNK)
            def _():
                pltpu.async_copy(y_hbm.at[gidx.at[i + 1]], rows1, sem1)

            pltpu.sync_copy(rows0, acc.at[didx.at[i]], add=True)

            @pl.when(i + 1 < _SCHUNK)
            def _():
                pltpu.make_async_copy(y_hbm.at[gidx.at[0]], rows1, sem1).wait()

                @pl.when(i + 2 < _SCHUNK)
                def _():
                    pltpu.async_copy(y_hbm.at[gidx.at[i + 2]], rows0, sem0)

                pltpu.sync_copy(rows1, acc.at[didx.at[i + 1]], add=True)

    plsc.subcore_barrier()

    pltpu.sync_copy(acc.at[pl.ds(s * _WSTRIDE, _WIN)],
                    out_hbm.at[pl.ds(c * _N + s * _WSTRIDE, _WIN)])


def _edge_aggregate(y_flat, g2d, d2d):
    mesh = plsc.VectorSubcoreMesh(core_axis_name="c", subcore_axis_name="s")
    kern = pl.kernel(
        _edge_body,
        out_type=jax.ShapeDtypeStruct((_NC * _N, _D), _F32),
        mesh=mesh,
        scratch_types=[
            pltpu.VMEM((_SCHUNK, _C), jnp.int32),
            pltpu.VMEM((_SCHUNK, _C), jnp.int32),
            pltpu.VMEM((_C, _D), _F32),
            pltpu.VMEM((_C, _D), _F32),
            pltpu.VMEM_SHARED((_N, _D), _F32),
            pltpu.SemaphoreType.DMA,
            pltpu.SemaphoreType.DMA,
        ],
    )
    return kern(y_flat, g2d, d2d)


def _combine_body(y_ref, p_ref, out_ref):
    out_ref[...] = y_ref[0] + p_ref[0] + p_ref[1]


def _combine(y5, partials, bn):
    nb = _N // bn
    return pl.pallas_call(
        _combine_body,
        grid=(nb,),
        in_specs=[
            pl.BlockSpec((1, bn, _D), lambda i: (_R, i, 0)),
            pl.BlockSpec((_NC, bn, _D), lambda i: (0, i, 0)),
        ],
        out_specs=pl.BlockSpec((bn, _D), lambda i: (i, 0)),
        out_shape=jax.ShapeDtypeStruct((_N, _D), _F32),
    )(y5, partials)


@jax.jit
def kernel(x, edge_index, etype, W_rel, w1_W, w1_b, mlp_W, mlp_b):
    src = edge_index[0]
    dst = edge_index[1]
    g = (etype * _N + src).reshape(_NW * _NSTAGE, _SCHUNK, _C)
    d2d = dst.reshape(_NW * _NSTAGE, _SCHUNK, _C)

    y5 = _project(x, W_rel, w1_W, mlp_W, w1_b, mlp_b, bn=1000)
    y_flat = y5.reshape((_R + 1) * _N, _D)
    partials = _edge_aggregate(y_flat, g, d2d)
    return _combine(y5, partials.reshape(_NC, _N, _D), bn=2000)


# speedup vs baseline: 11.5075x; 1.2981x over previous
import jax
import jax.numpy as jnp
from jax import lax
from jax.experimental import pallas as pl
from jax.experimental.pallas import tpu as pltpu
from jax.experimental.pallas import tpu_sc as plsc

_N = 10000
_E = 320000
_D = 128
_R = 4

_NC = 2
_NS = 16
_NW = _NC * _NS
_EPW = _E // _NW
_C = 80
_NCHUNK = _EPW // _C
_NSTAGE = 5
_SCHUNK = _NCHUNK // _NSTAGE
_WSTRIDE = 624
_WIN = 640

_F32 = jnp.float32
_HI = lax.Precision.HIGHEST


def _proj_body(x_ref, wrel_ref, w1w_ref, mlpw_ref, w1b_ref, mlpb_ref,
               y_ref, wc_ref, bc_ref):
    i = pl.program_id(0)

    @pl.when(i == 0)
    def _():
        for k in range(_R):
            wc_ref[k] = jax.lax.dot(wrel_ref[k], mlpw_ref[...],
                                    precision=_HI, preferred_element_type=_F32)
        wc_ref[_R] = jax.lax.dot(w1w_ref[...], mlpw_ref[...],
                                 precision=_HI, preferred_element_type=_F32)
        bc_ref[...] = jax.lax.dot(w1b_ref[...], mlpw_ref[...],
                                  precision=_HI, preferred_element_type=_F32) \
            + mlpb_ref[...]

    xb = x_ref[...]
    for k in range(_R + 1):
        yk = jax.lax.dot(xb, wc_ref[k], precision=_HI,
                         preferred_element_type=_F32)
        if k == _R:
            yk = yk + bc_ref[...]
        y_ref[k] = yk


def _project(x, W_rel, w1_W, mlp_W, w1_b, mlp_b, bn):
    nb = _N // bn
    return pl.pallas_call(
        _proj_body,
        grid=(nb,),
        in_specs=[
            pl.BlockSpec((bn, _D), lambda i: (i, 0)),
            pl.BlockSpec((_R, _D, _D), lambda i: (0, 0, 0)),
            pl.BlockSpec((_D, _D), lambda i: (0, 0)),
            pl.BlockSpec((_D, _D), lambda i: (0, 0)),
            pl.BlockSpec((1, _D), lambda i: (0, 0)),
            pl.BlockSpec((1, _D), lambda i: (0, 0)),
        ],
        out_specs=pl.BlockSpec((_R + 1, bn, _D), lambda i: (0, i, 0)),
        out_shape=jax.ShapeDtypeStruct((_R + 1, _N, _D), _F32),
        scratch_shapes=[
            pltpu.VMEM((_R + 1, _D, _D), _F32),
            pltpu.VMEM((1, _D), _F32),
        ],
    )(x, W_rel, w1_W, mlp_W, w1_b.reshape(1, _D), mlp_b.reshape(1, _D))


def _edge_body(y_hbm, g_hbm, d_hbm, out_hbm,
               gidx, didx, rows0, rows1, acc, sem0, sem1):
    c = lax.axis_index("c")
    s = lax.axis_index("s")
    wid = s * _NC + c

    zv = jnp.zeros((16,), _F32)

    @pl.loop(0, _C)
    def _(i):
        for k in range(_D // 16):
            rows0[i, pl.ds(k * 16, 16)] = zv

    for j in range(_WIN // _C):
        pltpu.sync_copy(rows0, acc.at[pl.ds(s * _WSTRIDE + j * _C, _C)])
    plsc.subcore_barrier()

    for h in range(_NSTAGE):
        pltpu.sync_copy(g_hbm.at[wid * _NSTAGE + h], gidx)
        pltpu.sync_copy(d_hbm.at[wid * _NSTAGE + h], didx)
        pltpu.async_copy(y_hbm.at[gidx.at[0]], rows0, sem0)

        @pl.loop(0, _SCHUNK, step=2)
        def _(i):
            pltpu.make_async_copy(y_hbm.at[gidx.at[0]], rows0, sem0).wait()

            @pl.when(i + 1 < _SCHUNK)
            def _():
                pltpu.async_copy(y_hbm.at[gidx.at[i + 1]], rows1, sem1)

            pltpu.sync_copy(rows0, acc.at[didx.at[i]], add=True)

            @pl.when(i + 1 < _SCHUNK)
            def _():
                pltpu.make_async_copy(y_hbm.at[gidx.at[0]], rows1, sem1).wait()

                @pl.when(i + 2 < _SCHUNK)
                def _():
                    pltpu.async_copy(y_hbm.at[gidx.at[i + 2]], rows0, sem0)

                pltpu.sync_copy(rows1, acc.at[didx.at[i + 1]], add=True)

    plsc.subcore_barrier()

    pltpu.sync_copy(acc.at[pl.ds(s * _WSTRIDE, _WIN)],
                    out_hbm.at[pl.ds(c * _N + s * _WSTRIDE, _WIN)])


def _edge_aggregate(y_flat, g2d, d2d):
    mesh = plsc.VectorSubcoreMesh(core_axis_name="c", subcore_axis_name="s")
    kern = pl.kernel(
        _edge_body,
        out_type=jax.ShapeDtypeStruct((_NC * _N, _D), _F32),
        mesh=mesh,
        scratch_types=[
            pltpu.VMEM((_SCHUNK, _C), jnp.int32),
            pltpu.VMEM((_SCHUNK, _C), jnp.int32),
            pltpu.VMEM((_C, _D), _F32),
            pltpu.VMEM((_C, _D), _F32),
            pltpu.VMEM_SHARED((_N, _D), _F32),
            pltpu.SemaphoreType.DMA,
            pltpu.SemaphoreType.DMA,
        ],
    )
    return kern(y_flat, g2d, d2d)


def _combine_body(y_ref, p_ref, out_ref):
    out_ref[...] = y_ref[0] + p_ref[0] + p_ref[1]


def _combine(y5, partials, bn):
    nb = _N // bn
    return pl.pallas_call(
        _combine_body,
        grid=(nb,),
        in_specs=[
            pl.BlockSpec((1, bn, _D), lambda i: (_R, i, 0)),
            pl.BlockSpec((_NC, bn, _D), lambda i: (0, i, 0)),
        ],
        out_specs=pl.BlockSpec((bn, _D), lambda i: (i, 0)),
        out_shape=jax.ShapeDtypeStruct((_N, _D), _F32),
    )(y5, partials)


@jax.jit
def kernel(x, edge_index, etype, W_rel, w1_W, w1_b, mlp_W, mlp_b):
    src = edge_index[0]
    dst = edge_index[1]
    g = (etype * _N + src).reshape(_NW * _NSTAGE, _SCHUNK, _C)
    d2d = dst.reshape(_NW * _NSTAGE, _SCHUNK, _C)

    y5 = _project(x, W_rel, w1_W, mlp_W, w1_b, mlp_b, bn=1000)
    y_flat = y5.reshape((_R + 1) * _N, _D)
    partials = _edge_aggregate(y_flat, g, d2d)
    return _combine(y5, partials.reshape(_NC, _N, _D), bn=2000)


# speedup vs baseline: 13.6053x; 1.1823x over previous
import jax
import jax.numpy as jnp
from jax import lax
from jax.experimental import pallas as pl
from jax.experimental.pallas import tpu as pltpu
from jax.experimental.pallas import tpu_sc as plsc

_N = 10000
_E = 320000
_D = 128
_R = 4

_NC = 2
_NS = 16
_NW = _NC * _NS
_EPW = _E // _NW
_C = 80
_NCHUNK = _EPW // _C
_NSTAGE = 5
_SCHUNK = _NCHUNK // _NSTAGE
_WSTRIDE = 624
_WIN = 640

_F32 = jnp.float32
_HI = lax.Precision.HIGHEST


def _proj_body(x_ref, wrel_ref, w1w_ref, mlpw_ref, w1b_ref, mlpb_ref,
               y_ref, wc_ref, bc_ref):
    i = pl.program_id(0)

    @pl.when(i == 0)
    def _():
        for k in range(_R):
            wc_ref[k] = jax.lax.dot(wrel_ref[k], mlpw_ref[...],
                                    precision=_HI, preferred_element_type=_F32)
        wc_ref[_R] = jax.lax.dot(w1w_ref[...], mlpw_ref[...],
                                 precision=_HI, preferred_element_type=_F32)
        bc_ref[...] = jax.lax.dot(w1b_ref[...], mlpw_ref[...],
                                  precision=_HI, preferred_element_type=_F32) \
            + mlpb_ref[...]

    xb = x_ref[...]
    for k in range(_R + 1):
        yk = jax.lax.dot(xb, wc_ref[k], precision=_HI,
                         preferred_element_type=_F32)
        if k == _R:
            yk = yk + bc_ref[...]
        y_ref[k] = yk


def _project(x, W_rel, w1_W, mlp_W, w1_b, mlp_b, bn):
    nb = _N // bn
    return pl.pallas_call(
        _proj_body,
        grid=(nb,),
        in_specs=[
            pl.BlockSpec((bn, _D), lambda i: (i, 0)),
            pl.BlockSpec((_R, _D, _D), lambda i: (0, 0, 0)),
            pl.BlockSpec((_D, _D), lambda i: (0, 0)),
            pl.BlockSpec((_D, _D), lambda i: (0, 0)),
            pl.BlockSpec((1, _D), lambda i: (0, 0)),
            pl.BlockSpec((1, _D), lambda i: (0, 0)),
        ],
        out_specs=pl.BlockSpec((_R + 1, bn, _D), lambda i: (0, i, 0)),
        out_shape=jax.ShapeDtypeStruct((_R + 1, _N, _D), _F32),
        scratch_shapes=[
            pltpu.VMEM((_R + 1, _D, _D), _F32),
            pltpu.VMEM((1, _D), _F32),
        ],
    )(x, W_rel, w1_W, mlp_W, w1_b.reshape(1, _D), mlp_b.reshape(1, _D))


def _edge_body(y_hbm, g_hbm, d_hbm, out_hbm,
               gidx, didx, rows0, rows1, rows2, rows3, acc,
               gs0, gs1, gs2, gs3, ss0, ss1, ss2, ss3):
    c = lax.axis_index("c")
    s = lax.axis_index("s")
    wid = s * _NC + c
    rows = (rows0, rows1, rows2, rows3)
    gsem = (gs0, gs1, gs2, gs3)
    ssem = (ss0, ss1, ss2, ss3)

    zv = jnp.zeros((16,), _F32)

    @pl.loop(0, _C)
    def _(i):
        for k in range(_D // 16):
            rows0[i, pl.ds(k * 16, 16)] = zv

    for j in range(_WIN // _C):
        pltpu.sync_copy(rows0, acc.at[pl.ds(s * _WSTRIDE + j * _C, _C)])
    plsc.subcore_barrier()

    for h in range(_NSTAGE):
        pltpu.sync_copy(g_hbm.at[wid * _NSTAGE + h], gidx)
        pltpu.sync_copy(d_hbm.at[wid * _NSTAGE + h], didx)
        pltpu.async_copy(y_hbm.at[gidx.at[0]], rows0, gs0)
        pltpu.async_copy(y_hbm.at[gidx.at[1]], rows1, gs1)

        @pl.loop(0, _SCHUNK, step=4)
        def _(i):
            for j in range(4):
                b = j
                b2 = (j + 2) % 4

                @pl.when(i + j < _SCHUNK)
                def _():
                    pltpu.make_async_copy(
                        y_hbm.at[gidx.at[0]], rows[b], gsem[b]).wait()
                    pltpu.async_copy(
                        rows[b], acc.at[didx.at[i + j]], ssem[b], add=True)

                    @pl.when(i + j + 2 < _SCHUNK)
                    def _():
                        @pl.when(i + j >= 2)
                        def _():
                            pltpu.make_async_copy(
                                rows[b2], acc.at[didx.at[0]], ssem[b2]).wait()

                        pltpu.async_copy(
                            y_hbm.at[gidx.at[i + j + 2]], rows[b2], gsem[b2])

        for q in range(_SCHUNK - 4, _SCHUNK):
            pltpu.make_async_copy(
                rows[q % 4], acc.at[didx.at[0]], ssem[q % 4]).wait()

    plsc.subcore_barrier()

    pltpu.sync_copy(acc.at[pl.ds(s * _WSTRIDE, _WIN)],
                    out_hbm.at[pl.ds(c * _N + s * _WSTRIDE, _WIN)])


def _edge_aggregate(y_flat, g2d, d2d):
    mesh = plsc.VectorSubcoreMesh(core_axis_name="c", subcore_axis_name="s")
    kern = pl.kernel(
        _edge_body,
        out_type=jax.ShapeDtypeStruct((_NC * _N, _D), _F32),
        mesh=mesh,
        scratch_types=[
            pltpu.VMEM((_SCHUNK, _C), jnp.int32),
            pltpu.VMEM((_SCHUNK, _C), jnp.int32),
            pltpu.VMEM((_C, _D), _F32),
            pltpu.VMEM((_C, _D), _F32),
            pltpu.VMEM((_C, _D), _F32),
            pltpu.VMEM((_C, _D), _F32),
            pltpu.VMEM_SHARED((_N, _D), _F32),
            pltpu.SemaphoreType.DMA,
            pltpu.SemaphoreType.DMA,
            pltpu.SemaphoreType.DMA,
            pltpu.SemaphoreType.DMA,
            pltpu.SemaphoreType.DMA,
            pltpu.SemaphoreType.DMA,
            pltpu.SemaphoreType.DMA,
            pltpu.SemaphoreType.DMA,
        ],
    )
    return kern(y_flat, g2d, d2d)


def _combine_body(y_ref, p_ref, out_ref):
    out_ref[...] = y_ref[0] + p_ref[0] + p_ref[1]


def _combine(y5, partials, bn):
    nb = _N // bn
    return pl.pallas_call(
        _combine_body,
        grid=(nb,),
        in_specs=[
            pl.BlockSpec((1, bn, _D), lambda i: (_R, i, 0)),
            pl.BlockSpec((_NC, bn, _D), lambda i: (0, i, 0)),
        ],
        out_specs=pl.BlockSpec((bn, _D), lambda i: (i, 0)),
        out_shape=jax.ShapeDtypeStruct((_N, _D), _F32),
    )(y5, partials)


@jax.jit
def kernel(x, edge_index, etype, W_rel, w1_W, w1_b, mlp_W, mlp_b):
    src = edge_index[0]
    dst = edge_index[1]
    g = (etype * _N + src).reshape(_NW * _NSTAGE, _SCHUNK, _C)
    d2d = dst.reshape(_NW * _NSTAGE, _SCHUNK, _C)

    y5 = _project(x, W_rel, w1_W, mlp_W, w1_b, mlp_b, bn=1000)
    y_flat = y5.reshape((_R + 1) * _N, _D)
    partials = _edge_aggregate(y_flat, g, d2d)
    return _combine(y5, partials.reshape(_NC, _N, _D), bn=2000)


# speedup vs baseline: 14.0442x; 1.0323x over previous
import jax
import jax.numpy as jnp
from jax import lax
from jax.experimental import pallas as pl
from jax.experimental.pallas import tpu as pltpu
from jax.experimental.pallas import tpu_sc as plsc

_N = 10000
_E = 320000
_D = 128
_R = 4

_NC = 2
_NS = 16
_NW = _NC * _NS
_EPW = _E // _NW
_C = 80
_NCHUNK = _EPW // _C
_NSTAGE = 5
_SCHUNK = _NCHUNK // _NSTAGE
_WSTRIDE = 624
_WIN = 640

_F32 = jnp.float32
_HI = lax.Precision.HIGHEST


def _proj_body(x_ref, wrel_ref, w1w_ref, mlpw_ref, w1b_ref, mlpb_ref,
               y_ref, wc_ref, bc_ref):
    i = pl.program_id(0)

    @pl.when(i == 0)
    def _():
        for k in range(_R):
            wc_ref[k] = jax.lax.dot(wrel_ref[k], mlpw_ref[...],
                                    precision=_HI, preferred_element_type=_F32)
        wc_ref[_R] = jax.lax.dot(w1w_ref[...], mlpw_ref[...],
                                 precision=_HI, preferred_element_type=_F32)
        bc_ref[...] = jax.lax.dot(w1b_ref[...], mlpw_ref[...],
                                  precision=_HI, preferred_element_type=_F32) \
            + mlpb_ref[...]

    xb = x_ref[...]
    for k in range(_R + 1):
        yk = jax.lax.dot(xb, wc_ref[k], preferred_element_type=_F32)
        if k == _R:
            yk = yk + bc_ref[...]
        y_ref[k] = yk


def _project(x, W_rel, w1_W, mlp_W, w1_b, mlp_b, bn):
    nb = _N // bn
    return pl.pallas_call(
        _proj_body,
        grid=(nb,),
        in_specs=[
            pl.BlockSpec((bn, _D), lambda i: (i, 0)),
            pl.BlockSpec((_R, _D, _D), lambda i: (0, 0, 0)),
            pl.BlockSpec((_D, _D), lambda i: (0, 0)),
            pl.BlockSpec((_D, _D), lambda i: (0, 0)),
            pl.BlockSpec((1, _D), lambda i: (0, 0)),
            pl.BlockSpec((1, _D), lambda i: (0, 0)),
        ],
        out_specs=pl.BlockSpec((_R + 1, bn, _D), lambda i: (0, i, 0)),
        out_shape=jax.ShapeDtypeStruct((_R + 1, _N, _D), _F32),
        scratch_shapes=[
            pltpu.VMEM((_R + 1, _D, _D), _F32),
            pltpu.VMEM((1, _D), _F32),
        ],
    )(x, W_rel, w1_W, mlp_W, w1_b.reshape(1, _D), mlp_b.reshape(1, _D))


def _eprep_body(ei_ref, et_ref, g_ref, d_ref):
    src = ei_ref[0:1, :]
    dst = ei_ref[1:2, :]
    g_ref[...] = et_ref[...] * _N + src
    d_ref[...] = dst


def _edge_prep(edge_index, etype):
    g, d = pl.pallas_call(
        _eprep_body,
        in_specs=[
            pl.BlockSpec((2, _E), lambda: (0, 0)),
            pl.BlockSpec((1, _E), lambda: (0, 0)),
        ],
        out_specs=[
            pl.BlockSpec((1, _E), lambda: (0, 0)),
            pl.BlockSpec((1, _E), lambda: (0, 0)),
        ],
        out_shape=[
            jax.ShapeDtypeStruct((1, _E), jnp.int32),
            jax.ShapeDtypeStruct((1, _E), jnp.int32),
        ],
    )(edge_index, etype.reshape(1, _E))
    shp = (_NW * _NSTAGE, _SCHUNK, _C)
    return g.reshape(shp), d.reshape(shp)


def _edge_body(y_hbm, g_hbm, d_hbm, out_hbm,
               gidx, didx, rows0, rows1, rows2, rows3, acc,
               gs0, gs1, gs2, gs3, ss0, ss1, ss2, ss3):
    c = lax.axis_index("c")
    s = lax.axis_index("s")
    wid = s * _NC + c
    rows = (rows0, rows1, rows2, rows3)
    gsem = (gs0, gs1, gs2, gs3)
    ssem = (ss0, ss1, ss2, ss3)

    @pl.when(c == 0)
    def _():
        pltpu.sync_copy(y_hbm.at[pl.ds(_R * _N + s * _WSTRIDE, _WIN)],
                        acc.at[pl.ds(s * _WSTRIDE, _WIN)])

    @pl.when(c == 1)
    def _():
        zv = jnp.zeros((16,), _F32)

        @pl.loop(0, _C)
        def _(i):
            for k in range(_D // 16):
                rows0[i, pl.ds(k * 16, 16)] = zv

        for j in range(_WIN // _C):
            pltpu.sync_copy(rows0, acc.at[pl.ds(s * _WSTRIDE + j * _C, _C)])

    plsc.subcore_barrier()

    for h in range(_NSTAGE):
        pltpu.sync_copy(g_hbm.at[wid * _NSTAGE + h], gidx)
        pltpu.sync_copy(d_hbm.at[wid * _NSTAGE + h], didx)
        pltpu.async_copy(y_hbm.at[gidx.at[0]], rows0, gs0)
        pltpu.async_copy(y_hbm.at[gidx.at[1]], rows1, gs1)

        @pl.loop(0, _SCHUNK, step=4)
        def _(i):
            for j in range(4):
                b = j
                b2 = (j + 2) % 4

                @pl.when(i + j < _SCHUNK)
                def _():
                    pltpu.make_async_copy(
                        y_hbm.at[gidx.at[0]], rows[b], gsem[b]).wait()
                    pltpu.async_copy(
                        rows[b], acc.at[didx.at[i + j]], ssem[b], add=True)

                    @pl.when(i + j + 2 < _SCHUNK)
                    def _():
                        @pl.when(i + j >= 2)
                        def _():
                            pltpu.make_async_copy(
                                rows[b2], acc.at[didx.at[0]], ssem[b2]).wait()

                        pltpu.async_copy(
                            y_hbm.at[gidx.at[i + j + 2]], rows[b2], gsem[b2])

        for q in range(_SCHUNK - 4, _SCHUNK):
            pltpu.make_async_copy(
                rows[q % 4], acc.at[didx.at[0]], ssem[q % 4]).wait()

    plsc.subcore_barrier()

    pltpu.sync_copy(acc.at[pl.ds(s * _WSTRIDE, _WIN)],
                    out_hbm.at[pl.ds(c * _N + s * _WSTRIDE, _WIN)])


def _edge_aggregate(y_flat, g2d, d2d):
    mesh = plsc.VectorSubcoreMesh(core_axis_name="c", subcore_axis_name="s")
    kern = pl.kernel(
        _edge_body,
        out_type=jax.ShapeDtypeStruct((_NC * _N, _D), _F32),
        mesh=mesh,
        scratch_types=[
            pltpu.VMEM((_SCHUNK, _C), jnp.int32),
            pltpu.VMEM((_SCHUNK, _C), jnp.int32),
            pltpu.VMEM((_C, _D), _F32),
            pltpu.VMEM((_C, _D), _F32),
            pltpu.VMEM((_C, _D), _F32),
            pltpu.VMEM((_C, _D), _F32),
            pltpu.VMEM_SHARED((_N, _D), _F32),
            pltpu.SemaphoreType.DMA,
            pltpu.SemaphoreType.DMA,
            pltpu.SemaphoreType.DMA,
            pltpu.SemaphoreType.DMA,
            pltpu.SemaphoreType.DMA,
            pltpu.SemaphoreType.DMA,
            pltpu.SemaphoreType.DMA,
            pltpu.SemaphoreType.DMA,
        ],
    )
    return kern(y_flat, g2d, d2d)


def _combine_body(p_ref, out_ref):
    out_ref[...] = p_ref[0] + p_ref[1]


def _combine(partials, bn):
    nb = _N // bn
    return pl.pallas_call(
        _combine_body,
        grid=(nb,),
        in_specs=[
            pl.BlockSpec((_NC, bn, _D), lambda i: (0, i, 0)),
        ],
        out_specs=pl.BlockSpec((bn, _D), lambda i: (i, 0)),
        out_shape=jax.ShapeDtypeStruct((_N, _D), _F32),
    )(partials)


@jax.jit
def kernel(x, edge_index, etype, W_rel, w1_W, w1_b, mlp_W, mlp_b):
    g, d2d = _edge_prep(edge_index, etype)
    y5 = _project(x, W_rel, w1_W, mlp_W, w1_b, mlp_b, bn=1000)
    y_flat = y5.reshape((_R + 1) * _N, _D)
    partials = _edge_aggregate(y_flat, g, d2d)
    return _combine(partials.reshape(_NC, _N, _D), bn=2000)


# speedup vs baseline: 14.8113x; 1.0546x over previous
import jax
import jax.numpy as jnp
from jax import lax
from jax.experimental import pallas as pl
from jax.experimental.pallas import tpu as pltpu
from jax.experimental.pallas import tpu_sc as plsc

_N = 10000
_E = 320000
_D = 128
_R = 4

_NC = 2
_NS = 16
_NW = _NC * _NS
_EPW = _E // _NW
_C = 80
_NCHUNK = _EPW // _C
_NSTAGE = 5
_SCHUNK = _NCHUNK // _NSTAGE
_WSTRIDE = 624
_WIN = 640

_F32 = jnp.float32
_HI = lax.Precision.HIGHEST


def _proj_body(x_ref, wrel_ref, w1w_ref, mlpw_ref, w1b_ref, mlpb_ref,
               y_ref, wc_ref, bc_ref):
    i = pl.program_id(0)

    @pl.when(i == 0)
    def _():
        for k in range(_R):
            wc_ref[k] = jax.lax.dot(wrel_ref[k], mlpw_ref[...],
                                    precision=_HI, preferred_element_type=_F32)
        wc_ref[_R] = jax.lax.dot(w1w_ref[...], mlpw_ref[...],
                                 precision=_HI, preferred_element_type=_F32)
        bc_ref[...] = jax.lax.dot(w1b_ref[...], mlpw_ref[...],
                                  precision=_HI, preferred_element_type=_F32) \
            + mlpb_ref[...]

    xb = x_ref[...]
    for k in range(_R + 1):
        yk = jax.lax.dot(xb, wc_ref[k], preferred_element_type=_F32)
        if k == _R:
            yk = yk + bc_ref[...]
        y_ref[k] = yk


def _project(x, W_rel, w1_W, mlp_W, w1_b, mlp_b, bn):
    nb = _N // bn
    return pl.pallas_call(
        _proj_body,
        grid=(nb,),
        in_specs=[
            pl.BlockSpec((bn, _D), lambda i: (i, 0)),
            pl.BlockSpec((_R, _D, _D), lambda i: (0, 0, 0)),
            pl.BlockSpec((_D, _D), lambda i: (0, 0)),
            pl.BlockSpec((_D, _D), lambda i: (0, 0)),
            pl.BlockSpec((1, _D), lambda i: (0, 0)),
            pl.BlockSpec((1, _D), lambda i: (0, 0)),
        ],
        out_specs=pl.BlockSpec((_R + 1, bn, _D), lambda i: (0, i, 0)),
        out_shape=jax.ShapeDtypeStruct((_R + 1, _N, _D), _F32),
        scratch_shapes=[
            pltpu.VMEM((_R + 1, _D, _D), _F32),
            pltpu.VMEM((1, _D), _F32),
        ],
    )(x, W_rel, w1_W, mlp_W, w1_b.reshape(1, _D), mlp_b.reshape(1, _D))


def _eprep_body(ei_ref, et_ref, g_ref, d_ref):
    src = ei_ref[0:1, :]
    dst = ei_ref[1:2, :]
    g_ref[...] = et_ref[...] * _N + src
    d_ref[...] = dst


def _edge_prep(edge_index, etype):
    g, d = pl.pallas_call(
        _eprep_body,
        in_specs=[
            pl.BlockSpec((2, _E), lambda: (0, 0)),
            pl.BlockSpec((1, _E), lambda: (0, 0)),
        ],
        out_specs=[
            pl.BlockSpec((1, _E), lambda: (0, 0)),
            pl.BlockSpec((1, _E), lambda: (0, 0)),
        ],
        out_shape=[
            jax.ShapeDtypeStruct((1, _E), jnp.int32),
            jax.ShapeDtypeStruct((1, _E), jnp.int32),
        ],
    )(edge_index, etype.reshape(1, _E))
    shp = (_NW * _NSTAGE, _SCHUNK, _C)
    return g.reshape(shp), d.reshape(shp)


def _edge_body(y_hbm, g_hbm, d_hbm, out_hbm,
               gidx, didx, rows0, rows1, rows2, rows3, acc,
               gs0, gs1, gs2, gs3, ss0, ss1, ss2, ss3):
    c = lax.axis_index("c")
    s = lax.axis_index("s")
    wid = s * _NC + c
    rows = (rows0, rows1, rows2, rows3)
    gsem = (gs0, gs1, gs2, gs3)
    ssem = (ss0, ss1, ss2, ss3)

    @pl.when(c == 0)
    def _():
        pltpu.sync_copy(y_hbm.at[pl.ds(_R * _N + s * _WSTRIDE, _WIN)],
                        acc.at[pl.ds(s * _WSTRIDE, _WIN)])

    @pl.when(c == 1)
    def _():
        zv = jnp.zeros((16,), _F32)

        @pl.loop(0, _C)
        def _(i):
            for k in range(_D // 16):
                rows0[i, pl.ds(k * 16, 16)] = zv

        for j in range(_WIN // _C):
            pltpu.sync_copy(rows0, acc.at[pl.ds(s * _WSTRIDE + j * _C, _C)])

    plsc.subcore_barrier()

    for h in range(_NSTAGE):
        pltpu.sync_copy(g_hbm.at[wid * _NSTAGE + h], gidx)
        pltpu.sync_copy(d_hbm.at[wid * _NSTAGE + h], didx)
        pltpu.async_copy(y_hbm.at[gidx.at[0]], rows0, gs0)
        pltpu.async_copy(y_hbm.at[gidx.at[1]], rows1, gs1)
        pltpu.async_copy(y_hbm.at[gidx.at[2]], rows2, gs2)

        @pl.loop(0, _SCHUNK, step=4)
        def _(i):
            for j in range(4):
                b = j
                b3 = (j + 3) % 4

                @pl.when(i + j < _SCHUNK)
                def _():
                    pltpu.make_async_copy(
                        y_hbm.at[gidx.at[0]], rows[b], gsem[b]).wait()
                    pltpu.async_copy(
                        rows[b], acc.at[didx.at[i + j]], ssem[b], add=True)

                    @pl.when(i + j + 3 < _SCHUNK)
                    def _():
                        @pl.when(i + j >= 1)
                        def _():
                            pltpu.make_async_copy(
                                rows[b3], acc.at[didx.at[0]], ssem[b3]).wait()

                        pltpu.async_copy(
                            y_hbm.at[gidx.at[i + j + 3]], rows[b3], gsem[b3])

        for q in range(_SCHUNK - 4, _SCHUNK):
            pltpu.make_async_copy(
                rows[q % 4], acc.at[didx.at[0]], ssem[q % 4]).wait()

    plsc.subcore_barrier()

    pltpu.sync_copy(acc.at[pl.ds(s * _WSTRIDE, _WIN)],
                    out_hbm.at[pl.ds(c * _N + s * _WSTRIDE, _WIN)])


def _edge_aggregate(y_flat, g2d, d2d):
    mesh = plsc.VectorSubcoreMesh(core_axis_name="c", subcore_axis_name="s")
    kern = pl.kernel(
        _edge_body,
        out_type=jax.ShapeDtypeStruct((_NC * _N, _D), _F32),
        mesh=mesh,
        scratch_types=[
            pltpu.VMEM((_SCHUNK, _C), jnp.int32),
            pltpu.VMEM((_SCHUNK, _C), jnp.int32),
            pltpu.VMEM((_C, _D), _F32),
            pltpu.VMEM((_C, _D), _F32),
            pltpu.VMEM((_C, _D), _F32),
            pltpu.VMEM((_C, _D), _F32),
            pltpu.VMEM_SHARED((_N, _D), _F32),
            pltpu.SemaphoreType.DMA,
            pltpu.SemaphoreType.DMA,
            pltpu.SemaphoreType.DMA,
            pltpu.SemaphoreType.DMA,
            pltpu.SemaphoreType.DMA,
            pltpu.SemaphoreType.DMA,
            pltpu.SemaphoreType.DMA,
            pltpu.SemaphoreType.DMA,
        ],
    )
    return kern(y_flat, g2d, d2d)


def _combine_body(p_ref, out_ref):
    out_ref[...] = p_ref[0] + p_ref[1]


def _combine(partials, bn):
    nb = _N // bn
    return pl.pallas_call(
        _combine_body,
        grid=(nb,),
        in_specs=[
            pl.BlockSpec((_NC, bn, _D), lambda i: (0, i, 0)),
        ],
        out_specs=pl.BlockSpec((bn, _D), lambda i: (i, 0)),
        out_shape=jax.ShapeDtypeStruct((_N, _D), _F32),
    )(partials)


@jax.jit
def kernel(x, edge_index, etype, W_rel, w1_W, w1_b, mlp_W, mlp_b):
    g, d2d = _edge_prep(edge_index, etype)
    y5 = _project(x, W_rel, w1_W, mlp_W, w1_b, mlp_b, bn=1000)
    y_flat = y5.reshape((_R + 1) * _N, _D)
    partials = _edge_aggregate(y_flat, g, d2d)
    return _combine(partials.reshape(_NC, _N, _D), bn=2000)


# speedup vs baseline: 16.3844x; 1.1062x over previous
import jax
import jax.numpy as jnp
from jax import lax
from jax.experimental import pallas as pl
from jax.experimental.pallas import tpu as pltpu
from jax.experimental.pallas import tpu_sc as plsc

_N = 10000
_E = 320000
_D = 128
_R = 4

_NC = 2
_NS = 16
_NW = _NC * _NS
_EPW = _E // _NW
_C = 80
_NCHUNK = _EPW // _C
_NSTAGE = 5
_SCHUNK = _NCHUNK // _NSTAGE
_WSTRIDE = 624
_WIN = 640

_F32 = jnp.float32
_HI = lax.Precision.HIGHEST


def _proj_body(x_ref, wrel_ref, w1w_ref, mlpw_ref, w1b_ref, mlpb_ref,
               y_ref, wc_ref, bc_ref):
    i = pl.program_id(0)

    @pl.when(i == 0)
    def _():
        for k in range(_R):
            wc_ref[k] = jax.lax.dot(wrel_ref[k], mlpw_ref[...],
                                    precision=_HI, preferred_element_type=_F32)
        wc_ref[_R] = jax.lax.dot(w1w_ref[...], mlpw_ref[...],
                                 precision=_HI, preferred_element_type=_F32)
        bc_ref[...] = jax.lax.dot(w1b_ref[...], mlpw_ref[...],
                                  precision=_HI, preferred_element_type=_F32) \
            + mlpb_ref[...]

    xb = x_ref[...]
    for k in range(_R + 1):
        yk = jax.lax.dot(xb, wc_ref[k], preferred_element_type=_F32)
        if k == _R:
            yk = yk + bc_ref[...]
        y_ref[k] = yk


def _project(x, W_rel, w1_W, mlp_W, w1_b, mlp_b, bn):
    nb = _N // bn
    return pl.pallas_call(
        _proj_body,
        grid=(nb,),
        in_specs=[
            pl.BlockSpec((bn, _D), lambda i: (i, 0)),
            pl.BlockSpec((_R, _D, _D), lambda i: (0, 0, 0)),
            pl.BlockSpec((_D, _D), lambda i: (0, 0)),
            pl.BlockSpec((_D, _D), lambda i: (0, 0)),
            pl.BlockSpec((1, _D), lambda i: (0, 0)),
            pl.BlockSpec((1, _D), lambda i: (0, 0)),
        ],
        out_specs=pl.BlockSpec((_R + 1, bn, _D), lambda i: (0, i, 0)),
        out_shape=jax.ShapeDtypeStruct((_R + 1, _N, _D), _F32),
        scratch_shapes=[
            pltpu.VMEM((_R + 1, _D, _D), _F32),
            pltpu.VMEM((1, _D), _F32),
        ],
    )(x, W_rel, w1_W, mlp_W, w1_b.reshape(1, _D), mlp_b.reshape(1, _D))


_ER = _E // 128


def _eprep_body(src_ref, et_ref, g_ref):
    g_ref[...] = et_ref[...] * _N + src_ref[...]


def _edge_prep(edge_index, etype):
    g = pl.pallas_call(
        _eprep_body,
        in_specs=[
            pl.BlockSpec((_ER, 128), lambda: (0, 0)),
            pl.BlockSpec((_ER, 128), lambda: (0, 0)),
        ],
        out_specs=pl.BlockSpec((_ER, 128), lambda: (0, 0)),
        out_shape=jax.ShapeDtypeStruct((_ER, 128), jnp.int32),
    )(edge_index[0].reshape(_ER, 128), etype.reshape(_ER, 128))
    d = edge_index[1].reshape(_NW * _NSTAGE, _SCHUNK, _C)
    return g.reshape(_E), d


def _edge_body(y_hbm, g_hbm, d_hbm, out_hbm,
               gidx, didx, rows0, rows1, rows2, rows3, acc,
               gs0, gs1, gs2, gs3, ss0, ss1, ss2, ss3):
    c = lax.axis_index("c")
    s = lax.axis_index("s")
    wid = s * _NC + c
    rows = (rows0, rows1, rows2, rows3)
    gsem = (gs0, gs1, gs2, gs3)
    ssem = (ss0, ss1, ss2, ss3)

    @pl.when(c == 0)
    def _():
        pltpu.sync_copy(y_hbm.at[pl.ds(_R * _N + s * _WSTRIDE, _WIN)],
                        acc.at[pl.ds(s * _WSTRIDE, _WIN)])

    @pl.when(c == 1)
    def _():
        zv = jnp.zeros((16,), _F32)

        @pl.loop(0, _C)
        def _(i):
            for k in range(_D // 16):
                rows0[i, pl.ds(k * 16, 16)] = zv

        for j in range(_WIN // _C):
            pltpu.sync_copy(rows0, acc.at[pl.ds(s * _WSTRIDE + j * _C, _C)])

    plsc.subcore_barrier()

    for h in range(_NSTAGE):
        pltpu.sync_copy(
            g_hbm.at[pl.ds((wid * _NSTAGE + h) * _SCHUNK * _C, _SCHUNK * _C)],
            gidx)
        pltpu.sync_copy(d_hbm.at[wid * _NSTAGE + h], didx)
        pltpu.async_copy(y_hbm.at[gidx.at[pl.ds(0, _C)]], rows0, gs0)
        pltpu.async_copy(y_hbm.at[gidx.at[pl.ds(_C, _C)]], rows1, gs1)
        pltpu.async_copy(y_hbm.at[gidx.at[pl.ds(2 * _C, _C)]], rows2, gs2)

        @pl.loop(0, _SCHUNK, step=4)
        def _(i):
            for j in range(4):
                b = j
                b3 = (j + 3) % 4

                @pl.when(i + j < _SCHUNK)
                def _():
                    pltpu.make_async_copy(
                        y_hbm.at[gidx.at[pl.ds(0, _C)]], rows[b], gsem[b]).wait()
                    pltpu.async_copy(
                        rows[b], acc.at[didx.at[i + j]], ssem[b], add=True)

                    @pl.when(i + j + 3 < _SCHUNK)
                    def _():
                        @pl.when(i + j >= 1)
                        def _():
                            pltpu.make_async_copy(
                                rows[b3], acc.at[didx.at[0]], ssem[b3]).wait()

                        pltpu.async_copy(
                            y_hbm.at[gidx.at[pl.ds((i + j + 3) * _C, _C)]], rows[b3], gsem[b3])

        for q in range(_SCHUNK - 4, _SCHUNK):
            pltpu.make_async_copy(
                rows[q % 4], acc.at[didx.at[0]], ssem[q % 4]).wait()

    plsc.subcore_barrier()

    pltpu.sync_copy(acc.at[pl.ds(s * _WSTRIDE, _WIN)],
                    out_hbm.at[pl.ds(c * _N + s * _WSTRIDE, _WIN)])


def _edge_aggregate(y_flat, g2d, d2d):
    mesh = plsc.VectorSubcoreMesh(core_axis_name="c", subcore_axis_name="s")
    kern = pl.kernel(
        _edge_body,
        out_type=jax.ShapeDtypeStruct((_NC * _N, _D), _F32),
        mesh=mesh,
        scratch_types=[
            pltpu.VMEM((_SCHUNK * _C,), jnp.int32),
            pltpu.VMEM((_SCHUNK, _C), jnp.int32),
            pltpu.VMEM((_C, _D), _F32),
            pltpu.VMEM((_C, _D), _F32),
            pltpu.VMEM((_C, _D), _F32),
            pltpu.VMEM((_C, _D), _F32),
            pltpu.VMEM_SHARED((_N, _D), _F32),
            pltpu.SemaphoreType.DMA,
            pltpu.SemaphoreType.DMA,
            pltpu.SemaphoreType.DMA,
            pltpu.SemaphoreType.DMA,
            pltpu.SemaphoreType.DMA,
            pltpu.SemaphoreType.DMA,
            pltpu.SemaphoreType.DMA,
            pltpu.SemaphoreType.DMA,
        ],
    )
    return kern(y_flat, g2d, d2d)


def _combine_body(p_ref, out_ref):
    out_ref[...] = p_ref[0] + p_ref[1]


def _combine(partials, bn):
    nb = _N // bn
    return pl.pallas_call(
        _combine_body,
        grid=(nb,),
        in_specs=[
            pl.BlockSpec((_NC, bn, _D), lambda i: (0, i, 0)),
        ],
        out_specs=pl.BlockSpec((bn, _D), lambda i: (i, 0)),
        out_shape=jax.ShapeDtypeStruct((_N, _D), _F32),
    )(partials)


@jax.jit
def kernel(x, edge_index, etype, W_rel, w1_W, w1_b, mlp_W, mlp_b):
    g, d2d = _edge_prep(edge_index, etype)
    y5 = _project(x, W_rel, w1_W, mlp_W, w1_b, mlp_b, bn=1000)
    y_flat = y5.reshape((_R + 1) * _N, _D)
    partials = _edge_aggregate(y_flat, g, d2d)
    return _combine(partials.reshape(_NC, _N, _D), bn=2000)


# speedup vs baseline: 16.5339x; 1.0091x over previous
import jax
import jax.numpy as jnp
from jax import lax
from jax.experimental import pallas as pl
from jax.experimental.pallas import tpu as pltpu
from jax.experimental.pallas import tpu_sc as plsc

_N = 10000
_E = 320000
_D = 128
_R = 4

_NC = 2
_NS = 16
_NW = _NC * _NS
_EPW = _E // _NW
_C = 80
_NCHUNK = _EPW // _C
_NSTAGE = 5
_SCHUNK = _NCHUNK // _NSTAGE
_WSTRIDE = 624
_WIN = 640

_F32 = jnp.float32
_HI = lax.Precision.HIGHEST


def _proj_body(x_ref, wrel_ref, w1w_ref, mlpw_ref, w1b_ref, mlpb_ref,
               y_ref, wc_ref, bc_ref):
    i = pl.program_id(0)

    @pl.when(i == 0)
    def _():
        for k in range(_R):
            wc_ref[k] = jax.lax.dot(wrel_ref[k], mlpw_ref[...],
                                    precision=_HI, preferred_element_type=_F32)
        wc_ref[_R] = jax.lax.dot(w1w_ref[...], mlpw_ref[...],
                                 precision=_HI, preferred_element_type=_F32)
        bc_ref[...] = jax.lax.dot(w1b_ref[...], mlpw_ref[...],
                                  precision=_HI, preferred_element_type=_F32) \
            + mlpb_ref[...]

    xb = x_ref[...]
    for k in range(_R + 1):
        yk = jax.lax.dot(xb, wc_ref[k], preferred_element_type=_F32)
        if k == _R:
            yk = yk + bc_ref[...]
        y_ref[k] = yk


def _project(x, W_rel, w1_W, mlp_W, w1_b, mlp_b, bn):
    nb = _N // bn
    return pl.pallas_call(
        _proj_body,
        grid=(nb,),
        in_specs=[
            pl.BlockSpec((bn, _D), lambda i: (i, 0)),
            pl.BlockSpec((_R, _D, _D), lambda i: (0, 0, 0)),
            pl.BlockSpec((_D, _D), lambda i: (0, 0)),
            pl.BlockSpec((_D, _D), lambda i: (0, 0)),
            pl.BlockSpec((1, _D), lambda i: (0, 0)),
            pl.BlockSpec((1, _D), lambda i: (0, 0)),
        ],
        out_specs=pl.BlockSpec((_R + 1, bn, _D), lambda i: (0, i, 0)),
        out_shape=jax.ShapeDtypeStruct((_R + 1, _N, _D), _F32),
        scratch_shapes=[
            pltpu.VMEM((_R + 1, _D, _D), _F32),
            pltpu.VMEM((1, _D), _F32),
        ],
    )(x, W_rel, w1_W, mlp_W, w1_b.reshape(1, _D), mlp_b.reshape(1, _D))


_ER = _E // 128


def _eprep_body(src_ref, et_ref, g_ref):
    g_ref[...] = et_ref[...] * _N + src_ref[...]


def _edge_prep(edge_index, etype):
    g = pl.pallas_call(
        _eprep_body,
        in_specs=[
            pl.BlockSpec((_ER, 128), lambda: (0, 0)),
            pl.BlockSpec((_ER, 128), lambda: (0, 0)),
        ],
        out_specs=pl.BlockSpec((_ER, 128), lambda: (0, 0)),
        out_shape=jax.ShapeDtypeStruct((_ER, 128), jnp.int32),
    )(edge_index[0].reshape(_ER, 128), etype.reshape(_ER, 128))
    return g.reshape(_E), edge_index[1]


def _edge_body(y_hbm, g_hbm, d_hbm, out_hbm,
               gidx, didx1, didx, rows0, rows1, rows2, rows3, acc,
               gs0, gs1, gs2, gs3, ss0, ss1, ss2, ss3):
    c = lax.axis_index("c")
    s = lax.axis_index("s")
    wid = s * _NC + c
    rows = (rows0, rows1, rows2, rows3)
    gsem = (gs0, gs1, gs2, gs3)
    ssem = (ss0, ss1, ss2, ss3)

    @pl.when(c == 0)
    def _():
        pltpu.sync_copy(y_hbm.at[pl.ds(_R * _N + s * _WSTRIDE, _WIN)],
                        acc.at[pl.ds(s * _WSTRIDE, _WIN)])

    @pl.when(c == 1)
    def _():
        zv = jnp.zeros((16,), _F32)

        @pl.loop(0, _C)
        def _(i):
            for k in range(_D // 16):
                rows0[i, pl.ds(k * 16, 16)] = zv

        for j in range(_WIN // _C):
            pltpu.sync_copy(rows0, acc.at[pl.ds(s * _WSTRIDE + j * _C, _C)])

    plsc.subcore_barrier()

    for h in range(_NSTAGE):
        piece = (wid * _NSTAGE + h) * _SCHUNK * _C
        pltpu.sync_copy(g_hbm.at[pl.ds(piece, _SCHUNK * _C)], gidx)
        pltpu.sync_copy(d_hbm.at[pl.ds(piece, _SCHUNK * _C)], didx1)

        @pl.loop(0, _SCHUNK)
        def _(i):
            for k in range(_C // 16):
                didx[i, pl.ds(k * 16, 16)] = didx1[pl.ds(i * _C + k * 16, 16)]
        pltpu.async_copy(y_hbm.at[gidx.at[pl.ds(0, _C)]], rows0, gs0)
        pltpu.async_copy(y_hbm.at[gidx.at[pl.ds(_C, _C)]], rows1, gs1)
        pltpu.async_copy(y_hbm.at[gidx.at[pl.ds(2 * _C, _C)]], rows2, gs2)

        @pl.loop(0, _SCHUNK, step=4)
        def _(i):
            for j in range(4):
                b = j
                b3 = (j + 3) % 4

                @pl.when(i + j < _SCHUNK)
                def _():
                    pltpu.make_async_copy(
                        y_hbm.at[gidx.at[pl.ds(0, _C)]], rows[b], gsem[b]).wait()
                    pltpu.async_copy(
                        rows[b], acc.at[didx.at[i + j]], ssem[b], add=True)

                    @pl.when(i + j + 3 < _SCHUNK)
                    def _():
                        @pl.when(i + j >= 1)
                        def _():
                            pltpu.make_async_copy(
                                rows[b3], acc.at[didx.at[0]], ssem[b3]).wait()

                        pltpu.async_copy(
                            y_hbm.at[gidx.at[pl.ds((i + j + 3) * _C, _C)]], rows[b3], gsem[b3])

        for q in range(_SCHUNK - 4, _SCHUNK):
            pltpu.make_async_copy(
                rows[q % 4], acc.at[didx.at[0]], ssem[q % 4]).wait()

    plsc.subcore_barrier()

    pltpu.sync_copy(acc.at[pl.ds(s * _WSTRIDE, _WIN)],
                    out_hbm.at[pl.ds(c * _N + s * _WSTRIDE, _WIN)])


def _edge_aggregate(y_flat, g2d, d2d):
    mesh = plsc.VectorSubcoreMesh(core_axis_name="c", subcore_axis_name="s")
    kern = pl.kernel(
        _edge_body,
        out_type=jax.ShapeDtypeStruct((_NC * _N, _D), _F32),
        mesh=mesh,
        scratch_types=[
            pltpu.VMEM((_SCHUNK * _C,), jnp.int32),
            pltpu.VMEM((_SCHUNK * _C,), jnp.int32),
            pltpu.VMEM((_SCHUNK, _C), jnp.int32),
            pltpu.VMEM((_C, _D), _F32),
            pltpu.VMEM((_C, _D), _F32),
            pltpu.VMEM((_C, _D), _F32),
            pltpu.VMEM((_C, _D), _F32),
            pltpu.VMEM_SHARED((_N, _D), _F32),
            pltpu.SemaphoreType.DMA,
            pltpu.SemaphoreType.DMA,
            pltpu.SemaphoreType.DMA,
            pltpu.SemaphoreType.DMA,
            pltpu.SemaphoreType.DMA,
            pltpu.SemaphoreType.DMA,
            pltpu.SemaphoreType.DMA,
            pltpu.SemaphoreType.DMA,
        ],
    )
    return kern(y_flat, g2d, d2d)


def _combine_body(p_ref, out_ref):
    out_ref[...] = p_ref[0] + p_ref[1]


def _combine(partials, bn):
    nb = _N // bn
    return pl.pallas_call(
        _combine_body,
        grid=(nb,),
        in_specs=[
            pl.BlockSpec((_NC, bn, _D), lambda i: (0, i, 0)),
        ],
        out_specs=pl.BlockSpec((bn, _D), lambda i: (i, 0)),
        out_shape=jax.ShapeDtypeStruct((_N, _D), _F32),
    )(partials)


@jax.jit
def kernel(x, edge_index, etype, W_rel, w1_W, w1_b, mlp_W, mlp_b):
    g, d2d = _edge_prep(edge_index, etype)
    y5 = _project(x, W_rel, w1_W, mlp_W, w1_b, mlp_b, bn=1000)
    y_flat = y5.reshape((_R + 1) * _N, _D)
    partials = _edge_aggregate(y_flat, g, d2d)
    return _combine(partials.reshape(_NC, _N, _D), bn=2000)
